# baseline (speedup 1.0000x reference)
import functools

import jax
import jax.numpy as jnp
from jax import lax
from jax.experimental import pallas as pl
from jax.experimental.pallas import tpu as pltpu
from jax.experimental.pallas import tpu_sc as plsc

INPUT_DIM = 2048
HIDDEN_DIM = 16384
TOPK = 64
BATCH = 2048

BH = 512

NC = 2
NS = 16
NW = NC * NS
ROWS_PER_W = BATCH // NW
L = 16
GCHUNK = 32


def _encode_body(x_ref, w_ref, b_ref, out_ref):
    xm = (x_ref[...] - b_ref[...][None, :]).astype(jnp.bfloat16)
    out_ref[...] = jax.lax.dot_general(
        xm, w_ref[...].astype(jnp.bfloat16),
        dimension_numbers=(((1,), (1,)), ((), ())),
        preferred_element_type=jnp.float32,
    )


def _encode(x, W_enc, bias):
    grid = (HIDDEN_DIM // BH,)
    return pl.pallas_call(
        _encode_body,
        grid=grid,
        in_specs=[
            pl.BlockSpec((BATCH, INPUT_DIM), lambda h: (0, 0)),
            pl.BlockSpec((BH, INPUT_DIM), lambda h: (h, 0)),
            pl.BlockSpec((INPUT_DIM,), lambda h: (0,)),
        ],
        out_specs=pl.BlockSpec((BATCH, BH), lambda h: (0, h)),
        out_shape=jax.ShapeDtypeStruct((BATCH, HIDDEN_DIM), jnp.float32),
    )(x, W_enc, bias)


def _sc_decode_body(vals_hbm, idx_hbm, w_hbm, bias_hbm, out_hbm,
                    vals_v, idx_v, bias_v, rows_v, acc_v, sem):
    wid = lax.axis_index("s") * NC + lax.axis_index("c")
    base_row = wid * ROWS_PER_W
    pltpu.sync_copy(bias_hbm, bias_v)

    def do_row(r, carry):
        row = base_row + r
        pltpu.sync_copy(idx_hbm.at[row], idx_v)
        pltpu.sync_copy(vals_hbm.at[row], vals_v)
        def cp(v, c):
            bias_vec = bias_v[pl.ds(v * L, L)]
            acc_v[pl.ds(v * L, L)] = bias_vec
            return c
        lax.fori_loop(0, INPUT_DIM // L, cp, 0, unroll=4)

        for half in range(TOPK // GCHUNK):
            pltpu.async_copy(
                w_hbm.at[idx_v.at[pl.ds(half * GCHUNK, GCHUNK)]],
                rows_v, sem).wait()

            for j in range(GCHUNK):
                jv = half * GCHUNK + j
                vv = vals_v[pl.ds((jv // L) * L, L)]
                s = jnp.maximum(vv[jv % L], 0.0)

                def do_v(v, c2, j=j, s=s):
                    seg = pl.ds(v * L, L)
                    plsc.addupdate(acc_v.at[seg], s * rows_v[j, seg])
                    return c2
                lax.fori_loop(0, INPUT_DIM // L, do_v, 0, unroll=4)

        pltpu.sync_copy(acc_v, out_hbm.at[row])
        return carry

    lax.fori_loop(0, ROWS_PER_W, do_row, 0)


def _sc_decode(vals, idx, W_enc, bias):
    mesh = plsc.VectorSubcoreMesh(core_axis_name="c", subcore_axis_name="s")
    k = pl.kernel(
        _sc_decode_body,
        mesh=mesh,
        out_type=jax.ShapeDtypeStruct((BATCH, INPUT_DIM), jnp.float32),
        scratch_types=[
            pltpu.VMEM((TOPK,), jnp.float32),
            pltpu.VMEM((TOPK,), jnp.int32),
            pltpu.VMEM((INPUT_DIM,), jnp.float32),
            pltpu.VMEM((GCHUNK, INPUT_DIM), jnp.float32),
            pltpu.VMEM((INPUT_DIM,), jnp.float32),
            pltpu.SemaphoreType.DMA,
        ],
    )
    return k(vals, idx, W_enc, bias)


def kernel(x, W_enc, W_dec, bias):
    pre = _encode(x, W_enc, bias)
    vals, idx = jax.lax.top_k(pre, TOPK)
    return _sc_decode(vals, idx, W_enc, bias)


# speedup vs baseline: 14.3283x; 14.3283x over previous
import jax
import jax.numpy as jnp
from jax import lax
from jax.experimental import pallas as pl
from jax.experimental.pallas import tpu as pltpu

INPUT_DIM = 2048
HIDDEN_DIM = 16384
TOPK = 64
BATCH = 2048

BH = 512
TB = 256
KB = 1024
DB = 1024


def _encode_body(x_ref, w_ref, b_ref, out_ref):
    xm = (x_ref[...] - b_ref[...][None, :]).astype(jnp.bfloat16)
    out_ref[...] = jax.lax.dot_general(
        xm, w_ref[...].astype(jnp.bfloat16),
        dimension_numbers=(((1,), (1,)), ((), ())),
        preferred_element_type=jnp.float32,
    )


def _encode(x, W_enc, bias):
    return pl.pallas_call(
        _encode_body,
        grid=(HIDDEN_DIM // BH,),
        in_specs=[
            pl.BlockSpec((BATCH, INPUT_DIM), lambda h: (0, 0)),
            pl.BlockSpec((BH, INPUT_DIM), lambda h: (h, 0)),
            pl.BlockSpec((INPUT_DIM,), lambda h: (0,)),
        ],
        out_specs=pl.BlockSpec((BATCH, BH), lambda h: (0, h)),
        out_shape=jax.ShapeDtypeStruct((BATCH, HIDDEN_DIM), jnp.float32),
    )(x, W_enc, bias)


def _u32_to_f32(t):
    neg = (t & jnp.uint32(0x80000000)) == 0
    bits = jnp.where(neg, ~t, t & jnp.uint32(0x7FFFFFFF))
    return lax.bitcast_convert_type(bits, jnp.float32)


def _threshold_body(pre_ref, t_ref):
    pre = pre_ref[...]

    def step(_, carry):
        lo, hi = carry
        mid = lo + ((hi - lo + jnp.uint32(1)) >> jnp.uint32(1))
        t_f = _u32_to_f32(mid)
        cnt = jnp.sum((pre >= t_f).astype(jnp.int32), axis=1, keepdims=True)
        take = cnt >= TOPK
        lo = jnp.where(take, mid, lo)
        hi = jnp.where(take, hi, mid - jnp.uint32(1))
        return lo, hi

    lo0 = jnp.zeros((TB, 1), jnp.uint32)
    hi0 = jnp.full((TB, 1), jnp.uint32(0xFFFFFFFE))
    lo, _ = lax.fori_loop(0, 32, step, (lo0, hi0))
    t_ref[...] = jnp.broadcast_to(_u32_to_f32(lo), (TB, 128))


def _threshold(pre):
    return pl.pallas_call(
        _threshold_body,
        grid=(BATCH // TB,),
        in_specs=[pl.BlockSpec((TB, HIDDEN_DIM), lambda b: (b, 0))],
        out_specs=pl.BlockSpec((TB, 128), lambda b: (b, 0)),
        out_shape=jax.ShapeDtypeStruct((BATCH, 128), jnp.float32),
    )(pre)


def _decode_body(pre_ref, t_ref, w_ref, b_ref, out_ref):
    k = pl.program_id(1)
    t = t_ref[...][:, :1]
    p = pre_ref[...]
    z = jnp.where(p >= t, jnp.maximum(p, 0.0), 0.0).astype(jnp.bfloat16)
    acc = jax.lax.dot_general(
        z, w_ref[...].astype(jnp.bfloat16),
        dimension_numbers=(((1,), (1,)), ((), ())),
        preferred_element_type=jnp.float32,
    )

    @pl.when(k == 0)
    def _():
        out_ref[...] = acc + b_ref[...][None, :]

    @pl.when(k > 0)
    def _():
        out_ref[...] += acc


def _masked_decode(pre, trep, W_dec, bias):
    return pl.pallas_call(
        _decode_body,
        grid=(BATCH // DB, HIDDEN_DIM // KB),
        in_specs=[
            pl.BlockSpec((DB, KB), lambda b, k: (b, k)),
            pl.BlockSpec((DB, 128), lambda b, k: (b, 0)),
            pl.BlockSpec((INPUT_DIM, KB), lambda b, k: (0, k)),
            pl.BlockSpec((INPUT_DIM,), lambda b, k: (0,)),
        ],
        out_specs=pl.BlockSpec((DB, INPUT_DIM), lambda b, k: (b, 0)),
        out_shape=jax.ShapeDtypeStruct((BATCH, INPUT_DIM), jnp.float32),
    )(pre, trep, W_dec, bias)


def kernel(x, W_enc, W_dec, bias):
    pre = _encode(x, W_enc, bias)
    trep = _threshold(pre)
    return _masked_decode(pre, trep, W_dec, bias)


# speedup vs baseline: 15.7510x; 1.0993x over previous
import jax
import jax.numpy as jnp
from jax import lax
from jax.experimental import pallas as pl
from jax.experimental.pallas import tpu as pltpu

INPUT_DIM = 2048
HIDDEN_DIM = 16384
TOPK = 64
BATCH = 2048

BH = 512
TB = 256
KB = 1024
DB = 1024


def _encode_body(x_ref, w_ref, b_ref, out_ref):
    xm = (x_ref[...] - b_ref[...][None, :]).astype(jnp.bfloat16)
    out_ref[...] = jax.lax.dot_general(
        xm, w_ref[...].astype(jnp.bfloat16),
        dimension_numbers=(((1,), (1,)), ((), ())),
        preferred_element_type=jnp.float32,
    )


def _encode(x, W_enc, bias):
    return pl.pallas_call(
        _encode_body,
        grid=(HIDDEN_DIM // BH,),
        in_specs=[
            pl.BlockSpec((BATCH, INPUT_DIM), lambda h: (0, 0)),
            pl.BlockSpec((BH, INPUT_DIM), lambda h: (h, 0)),
            pl.BlockSpec((INPUT_DIM,), lambda h: (0,)),
        ],
        out_specs=pl.BlockSpec((BATCH, BH), lambda h: (0, h)),
        out_shape=jax.ShapeDtypeStruct((BATCH, HIDDEN_DIM), jnp.float32),
    )(x, W_enc, bias)


def _u32_to_f32(t):
    neg = (t & jnp.uint32(0x80000000)) == 0
    bits = jnp.where(neg, ~t, t & jnp.uint32(0x7FFFFFFF))
    return lax.bitcast_convert_type(bits, jnp.float32)


def _f32_to_key(x):
    k = lax.bitcast_convert_type(x, jnp.int32)
    u = lax.bitcast_convert_type(k, jnp.uint32)
    return jnp.where(k < 0, ~u, u | jnp.uint32(0x80000000))


def _threshold_body(pre_ref, t_ref):
    pre = pre_ref[...]

    cm = pre[:, :128]
    for c in range(1, HIDDEN_DIM // 128):
        cm = jnp.maximum(cm, pre[:, c * 128:(c + 1) * 128])
    himax = _f32_to_key(jnp.max(cm, axis=1, keepdims=True))

    def step_cm(_, carry):
        lo, hi = carry
        mid = lo + ((hi - lo + jnp.uint32(1)) >> jnp.uint32(1))
        t_f = _u32_to_f32(mid)
        cnt = jnp.sum((cm >= t_f).astype(jnp.int32), axis=1, keepdims=True)
        take = cnt >= TOPK
        lo = jnp.where(take, mid, lo)
        hi = jnp.where(take, hi, mid - jnp.uint32(1))
        return lo, hi

    lo_cm, _ = lax.fori_loop(
        0, 32, step_cm, (jnp.zeros((TB, 1), jnp.uint32), himax))

    def cond(carry):
        lo, hi = carry
        return jnp.any(lo < hi)

    def step(carry):
        lo, hi = carry
        mid = lo + ((hi - lo + jnp.uint32(1)) >> jnp.uint32(1))
        t_f = _u32_to_f32(mid)
        cnt = jnp.sum((pre >= t_f).astype(jnp.int32), axis=1, keepdims=True)
        take = cnt >= TOPK
        lo = jnp.where(take, mid, lo)
        hi = jnp.where(take, hi, mid - jnp.uint32(1))
        return lo, hi

    lo, _ = lax.while_loop(cond, step, (lo_cm, himax))
    t_ref[...] = jnp.broadcast_to(_u32_to_f32(lo), (TB, 128))


def _threshold(pre):
    return pl.pallas_call(
        _threshold_body,
        grid=(BATCH // TB,),
        in_specs=[pl.BlockSpec((TB, HIDDEN_DIM), lambda b: (b, 0))],
        out_specs=pl.BlockSpec((TB, 128), lambda b: (b, 0)),
        out_shape=jax.ShapeDtypeStruct((BATCH, 128), jnp.float32),
    )(pre)


def _decode_body(pre_ref, t_ref, w_ref, b_ref, out_ref):
    k = pl.program_id(1)
    t = t_ref[...][:, :1]
    p = pre_ref[...]
    z = jnp.where(p >= t, jnp.maximum(p, 0.0), 0.0).astype(jnp.bfloat16)
    acc = jax.lax.dot_general(
        z, w_ref[...].astype(jnp.bfloat16),
        dimension_numbers=(((1,), (1,)), ((), ())),
        preferred_element_type=jnp.float32,
    )

    @pl.when(k == 0)
    def _():
        out_ref[...] = acc + b_ref[...][None, :]

    @pl.when(k > 0)
    def _():
        out_ref[...] += acc


def _masked_decode(pre, trep, W_dec, bias):
    return pl.pallas_call(
        _decode_body,
        grid=(BATCH // DB, HIDDEN_DIM // KB),
        in_specs=[
            pl.BlockSpec((DB, KB), lambda b, k: (b, k)),
            pl.BlockSpec((DB, 128), lambda b, k: (b, 0)),
            pl.BlockSpec((INPUT_DIM, KB), lambda b, k: (0, k)),
            pl.BlockSpec((INPUT_DIM,), lambda b, k: (0,)),
        ],
        out_specs=pl.BlockSpec((DB, INPUT_DIM), lambda b, k: (b, 0)),
        out_shape=jax.ShapeDtypeStruct((BATCH, INPUT_DIM), jnp.float32),
    )(pre, trep, W_dec, bias)


def kernel(x, W_enc, W_dec, bias):
    pre = _encode(x, W_enc, bias)
    trep = _threshold(pre)
    return _masked_decode(pre, trep, W_dec, bias)
